# baseline (speedup 1.0000x reference)
import functools

import jax
import jax.numpy as jnp
from jax import lax
from jax.experimental import pallas as pl
from jax.experimental.pallas import tpu as pltpu
from jax.experimental.pallas import tpu_sc as plsc

B, C = 1024, 100000
S = 64.0
M3 = 0.4

NC, NS = 2, 16
NW = NC * NS
RPW = B // NW
GPW = RPW // 8
CWC = 5888
KPG = 16
C1 = 2944
C2 = C - KPG * CWC - C1
T = GPW * KPG

_mesh = plsc.VectorSubcoreMesh(core_axis_name="c", subcore_axis_name="s")


@functools.partial(
    pl.kernel,
    out_type=jax.ShapeDtypeStruct((B, C), jnp.float32),
    mesh=_mesh,
    scratch_types=[
        pltpu.VMEM((8, CWC), jnp.float32),
        pltpu.VMEM((8, CWC), jnp.float32),
        pltpu.VMEM((8, C2), jnp.float32),
        pltpu.VMEM((RPW, 16), jnp.int32),
        pltpu.VMEM((RPW, 16), jnp.float32),
        pltpu.SemaphoreType.DMA,
        pltpu.SemaphoreType.DMA,
        pltpu.SemaphoreType.DMA,
        pltpu.SemaphoreType.DMA,
    ],
)
def _sc_margin_scale(logits_hbm, labs_hbm, margs_hbm, out_hbm,
                     buf0, buf1, tbuf, labs_v, margs_v,
                     lsem0, lsem1, ssem0, ssem1):
    wid = lax.axis_index("s") * NC + lax.axis_index("c")
    pltpu.sync_copy(labs_hbm.at[pl.ds(wid * RPW, RPW), :], labs_v)
    pltpu.sync_copy(margs_hbm.at[pl.ds(wid * RPW, RPW), :], margs_v)

    bufs = (buf0, buf1)
    lsems = (lsem0, lsem1)
    ssems = (ssem0, ssem1)

    def rows(g):
        return pl.ds((wid * GPW + g) * 8, 8)

    def src(t):
        return logits_hbm.at[rows(t // KPG), pl.ds((t % KPG) * CWC, CWC)]

    def dst(t):
        return out_hbm.at[rows(t // KPG), pl.ds((t % KPG) * CWC, CWC)]

    def scale(buf, ncols, j):
        @plsc.parallel_loop(0, ncols // 16, unroll=8)
        def _(i):
            sl = pl.ds(i * 16, 16)
            buf[j, sl] = buf[j, sl] * S

    def fixup(buf, g, c0, ncols):
        for j in range(8):
            lab = labs_v[g * 8 + j][0]
            marg = margs_v[g * 8 + j][0]
            pos = lab - c0

            @pl.when((pos >= 0) & (pos < ncols))
            def _():
                b16 = (pos // 16) * 16
                off = pos - b16
                sl = pl.ds(b16, 16)
                hit = lax.iota(jnp.int32, 16) == off
                buf[j, sl] = buf[j, sl] - jnp.where(hit, marg, 0.0)

    def process(t, buf):
        for j in range(8):
            scale(buf, CWC, j)
        fixup(buf, t // KPG, (t % KPG) * CWC, CWC)

    pltpu.make_async_copy(src(0), buf0, lsem0).start()

    def outer(kk, _):
        t0 = kk * 2
        for b in (0, 1):
            t = t0 + b
            buf, lsem, ssem = bufs[b], lsems[b], ssems[b]
            pltpu.make_async_copy(src(t), buf, lsem).wait()
            process(t, buf)
            pltpu.make_async_copy(buf, dst(t), ssem).start()
            ob = 1 - b
            tn = t + 1

            @pl.when(tn >= 2)
            def _():
                pltpu.make_async_copy(bufs[ob], dst(tn - 2), ssems[ob]).wait()

            @pl.when(tn < T)
            def _():
                pltpu.make_async_copy(src(tn), bufs[ob], lsems[ob]).start()

        return 0

    lax.fori_loop(0, T // 2, outer, 0)
    pltpu.make_async_copy(buf1, dst(T - 1), ssem1).wait()

    c1 = KPG * CWC
    c2 = c1 + C1

    def s1(g):
        return logits_hbm.at[rows(g), pl.ds(c1, C1)]

    def d1(g):
        return out_hbm.at[rows(g), pl.ds(c1, C1)]

    def p1buf(g):
        return bufs[g % 2].at[:, pl.ds(0, C1)]

    pltpu.make_async_copy(s1(0), p1buf(0), lsem0).start()
    pltpu.make_async_copy(s1(1), p1buf(1), lsem1).start()
    for g in range(GPW):
        b = g % 2
        pltpu.make_async_copy(s1(g), p1buf(g), lsems[b]).wait()
        buf = bufs[b]
        for j in range(8):
            scale(buf, C1, j)
        fixup(buf, g, c1, C1)
        pltpu.make_async_copy(p1buf(g), d1(g), ssems[b]).start()
        if g + 2 < GPW:
            pltpu.make_async_copy(p1buf(g), d1(g), ssems[b]).wait()
            pltpu.make_async_copy(s1(g + 2), p1buf(g + 2), lsems[b]).start()
    for g in (GPW - 2, GPW - 1):
        pltpu.make_async_copy(p1buf(g), d1(g), ssems[g % 2]).wait()

    for g in range(GPW):
        pltpu.sync_copy(logits_hbm.at[rows(g), pl.ds(c2, C2)], tbuf)
        for j in range(8):
            scale(tbuf, C2, j)
        fixup(tbuf, g, c2, C2)
        pltpu.sync_copy(tbuf, out_hbm.at[rows(g), pl.ds(c2, C2)])


def kernel(logits, labels):
    valid = labels != -1
    labs16 = jnp.broadcast_to(
        jnp.where(valid, labels, 0)[:, None], (B, 16)
    ).astype(jnp.int32)
    margs16 = jnp.broadcast_to(
        jnp.where(valid, M3 * S, 0.0)[:, None].astype(jnp.float32), (B, 16)
    )
    return _sc_margin_scale(logits, labs16, margs16)


# speedup vs baseline: 1.0576x; 1.0576x over previous
import functools

import jax
import jax.numpy as jnp
from jax import lax
from jax.experimental import pallas as pl
from jax.experimental.pallas import tpu as pltpu
from jax.experimental.pallas import tpu_sc as plsc

B, C = 1024, 100000
S = 64.0
M3 = 0.4

NC, NS = 2, 16
NW = NC * NS
RPW = B // NW
GPW = RPW // 8
CWC = 2944
KPG = 33
C2 = C - KPG * CWC
T = GPW * KPG

_mesh = plsc.VectorSubcoreMesh(core_axis_name="c", subcore_axis_name="s")


@functools.partial(
    pl.kernel,
    out_type=jax.ShapeDtypeStruct((B, C), jnp.float32),
    mesh=_mesh,
    scratch_types=[
        pltpu.VMEM((8, CWC), jnp.float32),
        pltpu.VMEM((8, CWC), jnp.float32),
        pltpu.VMEM((8, CWC), jnp.float32),
        pltpu.VMEM((8, CWC), jnp.float32),
        pltpu.VMEM((8, C2), jnp.float32),
        pltpu.VMEM((RPW, 16), jnp.int32),
        pltpu.VMEM((RPW, 16), jnp.float32),
        pltpu.SemaphoreType.DMA,
        pltpu.SemaphoreType.DMA,
        pltpu.SemaphoreType.DMA,
        pltpu.SemaphoreType.DMA,
    ],
)
def _sc_margin_scale(logits_hbm, labs_hbm, margs_hbm, out_hbm,
                     ibuf0, ibuf1, obuf0, obuf1, tbuf, labs_v, margs_v,
                     lsem0, lsem1, ssem0, ssem1):
    wid = lax.axis_index("s") * NC + lax.axis_index("c")
    pltpu.sync_copy(labs_hbm.at[pl.ds(wid * RPW, RPW), :], labs_v)
    pltpu.sync_copy(margs_hbm.at[pl.ds(wid * RPW, RPW), :], margs_v)

    ibufs = (ibuf0, ibuf1)
    obufs = (obuf0, obuf1)
    lsems = (lsem0, lsem1)
    ssems = (ssem0, ssem1)

    def rows(g):
        return pl.ds((wid * GPW + g) * 8, 8)

    def src(t):
        return logits_hbm.at[rows(t // KPG), pl.ds((t % KPG) * CWC, CWC)]

    def dst(t):
        return out_hbm.at[rows(t // KPG), pl.ds((t % KPG) * CWC, CWC)]

    def scale(ib, ob, ncols, j):
        @plsc.parallel_loop(0, ncols // 16, unroll=8)
        def _(i):
            sl = pl.ds(i * 16, 16)
            ob[j, sl] = ib[j, sl] * S

    def fixup(buf, g, c0, ncols):
        for j in range(8):
            lab = labs_v[g * 8 + j][0]
            marg = margs_v[g * 8 + j][0]
            pos = lab - c0

            @pl.when((pos >= 0) & (pos < ncols))
            def _():
                b16 = (pos // 16) * 16
                off = pos - b16
                sl = pl.ds(b16, 16)
                hit = lax.iota(jnp.int32, 16) == off
                buf[j, sl] = buf[j, sl] - jnp.where(hit, marg, 0.0)

    pltpu.make_async_copy(src(0), ibuf0, lsem0).start()

    def outer(kk, _):
        t0 = kk * 2
        for b in (0, 1):
            t = t0 + b
            ib, ob = ibufs[b], obufs[b]
            pltpu.make_async_copy(src(t), ib, lsems[b]).wait()

            @pl.when(t + 1 < T)
            def _():
                pltpu.make_async_copy(src(t + 1), ibufs[1 - b],
                                      lsems[1 - b]).start()

            @pl.when(t >= 2)
            def _():
                pltpu.make_async_copy(ob, dst(t - 2), ssems[b]).wait()

            for j in range(8):
                scale(ib, ob, CWC, j)
            fixup(ob, t // KPG, (t % KPG) * CWC, CWC)
            pltpu.make_async_copy(ob, dst(t), ssems[b]).start()

        return 0

    lax.fori_loop(0, T // 2, outer, 0)
    pltpu.make_async_copy(obuf0, dst(T - 2), ssem0).wait()
    pltpu.make_async_copy(obuf1, dst(T - 1), ssem1).wait()

    c2 = KPG * CWC
    for g in range(GPW):
        pltpu.sync_copy(logits_hbm.at[rows(g), pl.ds(c2, C2)], tbuf)
        for j in range(8):
            scale(tbuf, tbuf, C2, j)
        fixup(tbuf, g, c2, C2)
        pltpu.sync_copy(tbuf, out_hbm.at[rows(g), pl.ds(c2, C2)])


def kernel(logits, labels):
    valid = labels != -1
    labs16 = jnp.broadcast_to(
        jnp.where(valid, labels, 0)[:, None], (B, 16)
    ).astype(jnp.int32)
    margs16 = jnp.broadcast_to(
        jnp.where(valid, M3 * S, 0.0)[:, None].astype(jnp.float32), (B, 16)
    )
    return _sc_margin_scale(logits, labs16, margs16)


# speedup vs baseline: 4.2302x; 3.9997x over previous
import jax
import jax.numpy as jnp
from jax.experimental import pallas as pl

B, C = 1024, 100000
S = 64.0
M3 = 0.4
BLK = 2000


def _margin_scale_kernel(labs_ref, margs_ref, x_ref, o_ref):
    c0 = pl.program_id(0) * BLK
    x = x_ref[...]
    labs = labs_ref[...]
    margs = margs_ref[...]
    rowid = jax.lax.broadcasted_iota(jnp.int32, x.shape, 0) + c0
    hit = rowid == labs
    o_ref[...] = x * S - jnp.where(hit, margs, 0.0)


def kernel(logits, labels):
    valid = labels != -1
    labs_row = jnp.where(valid, labels, -2).astype(jnp.int32).reshape(1, B)
    margs_row = jnp.where(valid, M3 * S, 0.0).astype(jnp.float32).reshape(1, B)
    xT = jnp.swapaxes(logits, 0, 1)
    outT = pl.pallas_call(
        _margin_scale_kernel,
        grid=(C // BLK,),
        in_specs=[
            pl.BlockSpec((1, B), lambda i: (0, 0)),
            pl.BlockSpec((1, B), lambda i: (0, 0)),
            pl.BlockSpec((BLK, B), lambda i: (i, 0)),
        ],
        out_specs=pl.BlockSpec((BLK, B), lambda i: (i, 0)),
        out_shape=jax.ShapeDtypeStruct((C, B), jnp.float32),
    )(labs_row, margs_row, xT)
    return jnp.swapaxes(outT, 0, 1)
